# baseline (speedup 1.0000x reference)
import functools

import jax
import jax.numpy as jnp
from jax import lax
from jax.experimental import pallas as pl
from jax.experimental.pallas import tpu as pltpu
from jax.experimental.pallas import tpu_sc as plsc

N_NODES = 10000
D = 128
N_EDGES = 320000

NS = 16

C = 128
CPT = 160
EPT = C * CPT
EPAD = NS * EPT

NPAD = 10240
ROWS_PER_TILE = NPAD // NS


def _make_seg(with_counts):
    mesh = plsc.VectorSubcoreMesh(
        core_axis_name="c", subcore_axis_name="s", num_cores=1
    )
    out_type = [jax.ShapeDtypeStruct((NPAD, D), jnp.float32)]
    if with_counts:
        out_type.append(jax.ShapeDtypeStruct((NS, NPAD), jnp.float32))

    scratch = [
        pltpu.VMEM((C,), jnp.int32),
        pltpu.VMEM((C,), jnp.int32),
        pltpu.VMEM((C,), jnp.int32),
        pltpu.VMEM((C,), jnp.int32),
        pltpu.VMEM((C, D), jnp.float32),
        pltpu.VMEM((C, D), jnp.float32),
        pltpu.VMEM((16, D), jnp.float32),
        pltpu.VMEM_SHARED((NPAD, D), jnp.float32),
        pltpu.SemaphoreType.DMA,
        pltpu.SemaphoreType.DMA,
    ]
    if with_counts:
        scratch.append(pltpu.VMEM((NPAD,), jnp.float32))

    def body(vals_hbm, srcp_hbm, dstp_hbm, *rest):
        if with_counts:
            p_hbm, cnt_hbm = rest[0], rest[1]
            rest = rest[2:]
        else:
            p_hbm = rest[0]
            rest = rest[1:]
        sidx = rest[0:2]
        didx = rest[2:4]
        rows = rest[4:6]
        zrow = rest[6]
        acc = rest[7]
        sems = rest[8:10]
        cntv = rest[10] if with_counts else None

        sid = lax.axis_index("s")
        ebase = sid * EPT

        def zb(i, carry):
            for j in range(D // 16):
                zrow[i, pl.ds(j * 16, 16)] = jnp.zeros((16,), jnp.float32)
            return carry

        with jax.named_scope("zinit"):
            lax.fori_loop(0, 16, zb, 0)

        def zc(k, carry):
            pltpu.sync_copy(zrow, acc.at[pl.ds(sid * ROWS_PER_TILE + k * 16, 16)])
            return carry

        with jax.named_scope("zcopy"):
            lax.fori_loop(0, ROWS_PER_TILE // 16, zc, 0)

        if with_counts:
            def zcnt(i, carry):
                cntv[pl.ds(i * 16, 16)] = jnp.zeros((16,), jnp.float32)
                return carry

            with jax.named_scope("zcnt"):
                lax.fori_loop(0, NPAD // 16, zcnt, 0)

        plsc.subcore_barrier()

        ones16 = jnp.ones((16,), jnp.float32)

        def issue(b, ci):
            off = ebase + ci * C
            pltpu.sync_copy(srcp_hbm.at[pl.ds(off, C)], sidx[b])
            pltpu.sync_copy(dstp_hbm.at[pl.ds(off, C)], didx[b])
            pltpu.async_copy(vals_hbm.at[sidx[b]], rows[b], sems[b])

        def drain(b):
            pltpu.make_async_copy(vals_hbm.at[sidx[b]], rows[b], sems[b]).wait()
            pltpu.sync_copy(rows[b], acc.at[didx[b]], add=True)
            if with_counts:
                for j in range(C // 16):
                    dvec = didx[b][pl.ds(j * 16, 16)]
                    plsc.addupdate_scatter(cntv, [dvec], ones16)

        with jax.named_scope("mainloop"):
            issue(0, 0)
            issue(1, 1)

            def step(g, carry):
                for b in range(2):
                    drain(b)
                    issue(b, g * 2 + b + 2)
                return carry

            lax.fori_loop(0, CPT // 2 - 1, step, 0)
            drain(0)
            drain(1)

        with jax.named_scope("outbar"):
            plsc.subcore_barrier()

        with jax.named_scope("outcopy"):
            rbase = sid * ROWS_PER_TILE
            pltpu.sync_copy(
                acc.at[pl.ds(rbase, ROWS_PER_TILE)],
                p_hbm.at[pl.ds(rbase, ROWS_PER_TILE)],
            )
            if with_counts:
                pltpu.sync_copy(cntv, cnt_hbm.at[sid])

    return pl.kernel(
        body,
        out_type=tuple(out_type),
        mesh=mesh,
        scratch_types=scratch,
        compiler_params=pltpu.CompilerParams(needs_layout_passes=False),
    )


_SEG_COUNTS = _make_seg(True)
_SEG = _make_seg(False)

BLK = 1024


def _combine_body(p_ref, cnt_ref, v_ref, wl_ref, wr_ref, b_ref, o_ref, *, act):
    cnt = jnp.sum(cnt_ref[...], axis=0)
    recip = 1.0 / jnp.maximum(cnt, 1.0)
    agg = p_ref[...] * recip[:, None]
    r = (
        jnp.dot(agg, wl_ref[...], preferred_element_type=jnp.float32)
        + jnp.dot(v_ref[...], wr_ref[...], preferred_element_type=jnp.float32)
        + b_ref[...]
    )
    o_ref[...] = act(r)


def _make_combine(act):
    return pl.pallas_call(
        functools.partial(_combine_body, act=act),
        grid=(NPAD // BLK,),
        in_specs=[
            pl.BlockSpec((BLK, D), lambda i: (i, 0)),
            pl.BlockSpec((NS, BLK), lambda i: (0, i)),
            pl.BlockSpec((BLK, D), lambda i: (i, 0)),
            pl.BlockSpec((D, D), lambda i: (0, 0)),
            pl.BlockSpec((D, D), lambda i: (0, 0)),
            pl.BlockSpec((1, D), lambda i: (0, 0)),
        ],
        out_specs=pl.BlockSpec((BLK, D), lambda i: (i, 0)),
        out_shape=jax.ShapeDtypeStruct((NPAD, D), jnp.float32),
    )


_COMBINE_RELU = _make_combine(jax.nn.relu)
_COMBINE_SIGMOID = _make_combine(jax.nn.sigmoid)


def kernel(x, edge_index, W1_l, b1, W1_r, W2_l, b2, W2_r):
    src = edge_index[0].astype(jnp.int32)
    dst = edge_index[1].astype(jnp.int32)
    pad_e = EPAD - N_EDGES
    srcp = jnp.concatenate([src, jnp.zeros((pad_e,), jnp.int32)])
    dstp = jnp.concatenate([dst, jnp.full((pad_e,), NPAD - 1, jnp.int32)])
    xp = jnp.concatenate(
        [x.astype(jnp.float32), jnp.zeros((NPAD - N_NODES, D), jnp.float32)]
    )

    p1, cnt = _SEG_COUNTS(xp, srcp, dstp)
    h = _COMBINE_RELU(p1, cnt, xp, W1_l.T, W1_r.T, b1.reshape(1, D))
    p2 = _SEG(h, srcp, dstp)
    if isinstance(p2, (list, tuple)):
        p2 = p2[0]
    out = _COMBINE_SIGMOID(p2, cnt, h, W2_l.T, W2_r.T, b2.reshape(1, D))
    return out[:N_NODES]


# speedup vs baseline: 1.7856x; 1.7856x over previous
import functools

import jax
import jax.numpy as jnp
from jax import lax
from jax.experimental import pallas as pl
from jax.experimental.pallas import tpu as pltpu
from jax.experimental.pallas import tpu_sc as plsc

N_NODES = 10000
D = 128
N_EDGES = 320000

NS = 16

C = 128
CPT = 160
EPT = C * CPT
EPAD = NS * EPT

NPAD = 10240
ROWS_PER_TILE = NPAD // NS


def _make_seg(with_counts):
    mesh = plsc.VectorSubcoreMesh(
        core_axis_name="c", subcore_axis_name="s", num_cores=1
    )
    out_type = [jax.ShapeDtypeStruct((NPAD, D), jnp.float32)]
    if with_counts:
        out_type.append(jax.ShapeDtypeStruct((NS, NPAD), jnp.float32))

    scratch = [
        pltpu.VMEM((C,), jnp.int32),
        pltpu.VMEM((C,), jnp.int32),
        pltpu.VMEM((C,), jnp.int32),
        pltpu.VMEM((C,), jnp.int32),
        pltpu.VMEM((C, D), jnp.float32),
        pltpu.VMEM((C, D), jnp.float32),
        pltpu.VMEM((16, D), jnp.float32),
        pltpu.VMEM_SHARED((NPAD, D), jnp.float32),
        pltpu.SemaphoreType.DMA,
        pltpu.SemaphoreType.DMA,
    ]
    if with_counts:
        scratch.append(pltpu.VMEM((NPAD,), jnp.float32))

    def body(vals_hbm, srcp_hbm, dstp_hbm, *rest):
        if with_counts:
            p_hbm, cnt_hbm = rest[0], rest[1]
            rest = rest[2:]
        else:
            p_hbm = rest[0]
            rest = rest[1:]
        sidx = rest[0:2]
        didx = rest[2:4]
        rows = rest[4:6]
        zrow = rest[6]
        acc = rest[7]
        sems = rest[8:10]
        cntv = rest[10] if with_counts else None

        sid = lax.axis_index("s")
        ebase = sid * EPT

        def zb(i, carry):
            for j in range(D // 16):
                zrow[i, pl.ds(j * 16, 16)] = jnp.zeros((16,), jnp.float32)
            return carry

        with jax.named_scope("zinit"):
            lax.fori_loop(0, 16, zb, 0)

        def zc(k, carry):
            pltpu.sync_copy(zrow, acc.at[pl.ds(sid * ROWS_PER_TILE + k * 16, 16)])
            return carry

        with jax.named_scope("zcopy"):
            lax.fori_loop(0, ROWS_PER_TILE // 16, zc, 0)

        if with_counts:
            def zcnt(i, carry):
                cntv[pl.ds(i * 16, 16)] = jnp.zeros((16,), jnp.float32)
                return carry

            with jax.named_scope("zcnt"):
                lax.fori_loop(0, NPAD // 16, zcnt, 0)

        plsc.subcore_barrier()

        ones16 = jnp.ones((16,), jnp.float32)

        def issue(b, ci):
            off = ebase + ci * C
            pltpu.sync_copy(srcp_hbm.at[pl.ds(off, C)], sidx[b])
            pltpu.sync_copy(dstp_hbm.at[pl.ds(off, C)], didx[b])
            pltpu.async_copy(vals_hbm.at[sidx[b]], rows[b], sems[b])

        def drain(b):
            pltpu.make_async_copy(vals_hbm.at[sidx[b]], rows[b], sems[b]).wait()
            pltpu.sync_copy(rows[b], acc.at[didx[b]], add=True)
            if with_counts:
                for j in range(C // 16):
                    dvec = didx[b][pl.ds(j * 16, 16)]
                    plsc.addupdate_scatter(cntv, [dvec], ones16)

        with jax.named_scope("mainloop"):
            issue(0, 0)
            issue(1, 1)

            def step(g, carry):
                for b in range(2):
                    drain(b)
                    issue(b, g * 2 + b + 2)
                return carry

            lax.fori_loop(0, CPT // 2 - 1, step, 0)
            drain(0)
            drain(1)

        with jax.named_scope("outbar"):
            plsc.subcore_barrier()

        with jax.named_scope("outcopy"):
            rbase = sid * ROWS_PER_TILE
            pltpu.sync_copy(
                acc.at[pl.ds(rbase, ROWS_PER_TILE)],
                p_hbm.at[pl.ds(rbase, ROWS_PER_TILE)],
            )
            if with_counts:
                pltpu.sync_copy(cntv, cnt_hbm.at[sid])

    return pl.kernel(
        body,
        out_type=tuple(out_type),
        mesh=mesh,
        scratch_types=scratch,
        compiler_params=pltpu.CompilerParams(needs_layout_passes=False),
    )


_SEG_COUNTS = _make_seg(True)
_SEG = _make_seg(False)

BLK = 1024


def _combine_body(p_ref, cnt_ref, v_ref, wl_ref, wr_ref, b_ref, o_ref, *, act):
    cnt = jnp.sum(cnt_ref[...], axis=0)
    recip = 1.0 / jnp.maximum(cnt, 1.0)
    agg = p_ref[...] * recip[:, None]
    r = (
        jnp.dot(agg, wl_ref[...], preferred_element_type=jnp.float32)
        + jnp.dot(v_ref[...], wr_ref[...], preferred_element_type=jnp.float32)
        + b_ref[...]
    )
    o_ref[...] = act(r)


def _make_combine(act):
    return pl.pallas_call(
        functools.partial(_combine_body, act=act),
        grid=(NPAD // BLK,),
        in_specs=[
            pl.BlockSpec((BLK, D), lambda i: (i, 0)),
            pl.BlockSpec((NS, BLK), lambda i: (0, i)),
            pl.BlockSpec((BLK, D), lambda i: (i, 0)),
            pl.BlockSpec((D, D), lambda i: (0, 0)),
            pl.BlockSpec((D, D), lambda i: (0, 0)),
            pl.BlockSpec((1, D), lambda i: (0, 0)),
        ],
        out_specs=pl.BlockSpec((BLK, D), lambda i: (i, 0)),
        out_shape=jax.ShapeDtypeStruct((NPAD, D), jnp.float32),
    )


_COMBINE_RELU = _make_combine(jax.nn.relu)
_COMBINE_SIGMOID = _make_combine(jax.nn.sigmoid)


def kernel(x, edge_index, W1_l, b1, W1_r, W2_l, b2, W2_r):
    src = edge_index[0].astype(jnp.int32)
    dst = edge_index[1].astype(jnp.int32)
    pad_e = EPAD - N_EDGES
    pad_iota = jnp.arange(pad_e, dtype=jnp.int32)
    srcp = jnp.concatenate([src, pad_iota % N_NODES])
    dstp = jnp.concatenate([dst, N_NODES + pad_iota % (NPAD - N_NODES)])
    xp = jnp.concatenate(
        [x.astype(jnp.float32), jnp.zeros((NPAD - N_NODES, D), jnp.float32)]
    )

    p1, cnt = _SEG_COUNTS(xp, srcp, dstp)
    h = _COMBINE_RELU(p1, cnt, xp, W1_l.T, W1_r.T, b1.reshape(1, D))
    p2 = _SEG(h, srcp, dstp)
    if isinstance(p2, (list, tuple)):
        p2 = p2[0]
    out = _COMBINE_SIGMOID(p2, cnt, h, W2_l.T, W2_r.T, b2.reshape(1, D))
    return out[:N_NODES]


# speedup vs baseline: 3.0311x; 1.6975x over previous
import functools

import jax
import jax.numpy as jnp
from jax import lax
from jax.experimental import pallas as pl
from jax.experimental.pallas import tpu as pltpu
from jax.experimental.pallas import tpu_sc as plsc

N_NODES = 10000
D = 128
N_EDGES = 320000

NC = 2
NS = 16
NW = NC * NS

C = 128
CPT0 = 80
CPT1 = 80
NCHUNKS = NS * (CPT0 + CPT1)
EPAD = NCHUNKS * C

NPAD = 10240
ROWS_PER_TILE = NPAD // NS


def _make_seg(with_counts):
    mesh = plsc.VectorSubcoreMesh(core_axis_name="c", subcore_axis_name="s")
    out_type = [jax.ShapeDtypeStruct((NC * NPAD, D), jnp.float32)]
    if with_counts:
        out_type.append(jax.ShapeDtypeStruct((NW, NPAD), jnp.float32))

    scratch = [
        pltpu.VMEM((C,), jnp.int32),
        pltpu.VMEM((C,), jnp.int32),
        pltpu.VMEM((C,), jnp.int32),
        pltpu.VMEM((C,), jnp.int32),
        pltpu.VMEM((C, D), jnp.float32),
        pltpu.VMEM((C, D), jnp.float32),
        pltpu.VMEM((16, D), jnp.float32),
        pltpu.VMEM_SHARED((NPAD, D), jnp.float32),
        pltpu.SemaphoreType.DMA,
        pltpu.SemaphoreType.DMA,
    ]
    if with_counts:
        scratch.append(pltpu.VMEM((NPAD,), jnp.float32))

    def body(vals_hbm, srcp_hbm, dstp_hbm, *rest):
        if with_counts:
            p_hbm, cnt_hbm = rest[0], rest[1]
            rest = rest[2:]
        else:
            p_hbm = rest[0]
            rest = rest[1:]
        sidx = rest[0:2]
        didx = rest[2:4]
        rows = rest[4:6]
        zrow = rest[6]
        acc = rest[7]
        sems = rest[8:10]
        cntv = rest[10] if with_counts else None

        cid = lax.axis_index("c")
        sid = lax.axis_index("s")
        wid = sid * NC + cid
        ebase = jnp.where(
            cid == 0,
            sid * (CPT0 * C),
            NS * (CPT0 * C) + sid * (CPT1 * C),
        )
        nsteps = jnp.where(cid == 0, CPT0 // 2 - 1, CPT1 // 2 - 1)

        def zb(i, carry):
            for j in range(D // 16):
                zrow[i, pl.ds(j * 16, 16)] = jnp.zeros((16,), jnp.float32)
            return carry

        with jax.named_scope("zinit"):
            lax.fori_loop(0, 16, zb, 0)

        def zc(k, carry):
            pltpu.sync_copy(zrow, acc.at[pl.ds(sid * ROWS_PER_TILE + k * 16, 16)])
            return carry

        with jax.named_scope("zcopy"):
            lax.fori_loop(0, ROWS_PER_TILE // 16, zc, 0)

        if with_counts:
            def zcnt(i, carry):
                cntv[pl.ds(i * 16, 16)] = jnp.zeros((16,), jnp.float32)
                return carry

            with jax.named_scope("zcnt"):
                lax.fori_loop(0, NPAD // 16, zcnt, 0)

        plsc.subcore_barrier()

        ones16 = jnp.ones((16,), jnp.float32)

        def issue(b, ci):
            off = ebase + ci * C
            pltpu.sync_copy(srcp_hbm.at[pl.ds(off, C)], sidx[b])
            pltpu.sync_copy(dstp_hbm.at[pl.ds(off, C)], didx[b])
            pltpu.async_copy(vals_hbm.at[sidx[b]], rows[b], sems[b])

        def drain(b):
            pltpu.make_async_copy(vals_hbm.at[sidx[b]], rows[b], sems[b]).wait()
            pltpu.sync_copy(rows[b], acc.at[didx[b]], add=True)
            if with_counts:
                for j in range(C // 16):
                    dvec = didx[b][pl.ds(j * 16, 16)]
                    plsc.addupdate_scatter(cntv, [dvec], ones16)

        with jax.named_scope("mainloop"):
            issue(0, 0)
            issue(1, 1)

            def step(g, carry):
                for b in range(2):
                    drain(b)
                    issue(b, g * 2 + b + 2)
                return carry

            lax.fori_loop(0, nsteps, step, 0)
            drain(0)
            drain(1)

        with jax.named_scope("outbar"):
            plsc.subcore_barrier()

        with jax.named_scope("outcopy"):
            rbase = sid * ROWS_PER_TILE
            pltpu.sync_copy(
                acc.at[pl.ds(rbase, ROWS_PER_TILE)],
                p_hbm.at[pl.ds(cid * NPAD + rbase, ROWS_PER_TILE)],
            )
            if with_counts:
                pltpu.sync_copy(cntv, cnt_hbm.at[wid])

    return pl.kernel(
        body,
        out_type=tuple(out_type),
        mesh=mesh,
        scratch_types=scratch,
        compiler_params=pltpu.CompilerParams(needs_layout_passes=False),
    )


_SEG_COUNTS = _make_seg(True)
_SEG = _make_seg(False)

BLK = 1024


def _combine_body(p0_ref, p1_ref, cnt_ref, v_ref, wl_ref, wr_ref, b_ref, o_ref, *, act):
    cnt = jnp.sum(cnt_ref[...], axis=0)
    recip = 1.0 / jnp.maximum(cnt, 1.0)
    agg = (p0_ref[...] + p1_ref[...]) * recip[:, None]
    r = (
        jnp.dot(agg, wl_ref[...], preferred_element_type=jnp.float32)
        + jnp.dot(v_ref[...], wr_ref[...], preferred_element_type=jnp.float32)
        + b_ref[...]
    )
    o_ref[...] = act(r)


def _make_combine(act):
    return pl.pallas_call(
        functools.partial(_combine_body, act=act),
        grid=(NPAD // BLK,),
        in_specs=[
            pl.BlockSpec((BLK, D), lambda i: (i, 0)),
            pl.BlockSpec((BLK, D), lambda i: (i + NPAD // BLK, 0)),
            pl.BlockSpec((NW, BLK), lambda i: (0, i)),
            pl.BlockSpec((BLK, D), lambda i: (i, 0)),
            pl.BlockSpec((D, D), lambda i: (0, 0)),
            pl.BlockSpec((D, D), lambda i: (0, 0)),
            pl.BlockSpec((1, D), lambda i: (0, 0)),
        ],
        out_specs=pl.BlockSpec((BLK, D), lambda i: (i, 0)),
        out_shape=jax.ShapeDtypeStruct((NPAD, D), jnp.float32),
    )


_COMBINE_RELU = _make_combine(jax.nn.relu)
_COMBINE_SIGMOID = _make_combine(jax.nn.sigmoid)


def kernel(x, edge_index, W1_l, b1, W1_r, W2_l, b2, W2_r):
    src = edge_index[0].astype(jnp.int32)
    dst = edge_index[1].astype(jnp.int32)
    pad_e = EPAD - N_EDGES
    pad_iota = jnp.arange(pad_e, dtype=jnp.int32)
    srcp = jnp.concatenate([src, pad_iota % N_NODES])
    dstp = jnp.concatenate([dst, N_NODES + pad_iota % (NPAD - N_NODES)])
    xp = jnp.concatenate(
        [x.astype(jnp.float32), jnp.zeros((NPAD - N_NODES, D), jnp.float32)]
    )

    p1, cnt = _SEG_COUNTS(xp, srcp, dstp)
    h = _COMBINE_RELU(p1, p1, cnt, xp, W1_l.T, W1_r.T, b1.reshape(1, D))
    p2 = _SEG(h, srcp, dstp)
    if isinstance(p2, (list, tuple)):
        p2 = p2[0]
    out = _COMBINE_SIGMOID(p2, p2, cnt, h, W2_l.T, W2_r.T, b2.reshape(1, D))
    return out[:N_NODES]


# speedup vs baseline: 3.7855x; 1.2489x over previous
import functools

import jax
import jax.numpy as jnp
from jax import lax
from jax.experimental import pallas as pl
from jax.experimental.pallas import tpu as pltpu
from jax.experimental.pallas import tpu_sc as plsc

N_NODES = 10000
D = 128
N_EDGES = 320000

NC = 2
NS = 16
NW = NC * NS

C = 128
CPT = 80
NCHUNKS = NW * CPT
EPAD = NCHUNKS * C
NBUF = 2

NPAD = 10240
ROWS_PER_TILE = NPAD // NS


HALF = CPT // 2


def _make_seg():
    mesh = plsc.VectorSubcoreMesh(core_axis_name="c", subcore_axis_name="s")

    scratch = (
        [pltpu.VMEM((HALF, C), jnp.int32)] * 2
        + [pltpu.VMEM((C, D), jnp.float32)] * NBUF
        + [pltpu.VMEM((16, D), jnp.float32)]
        + [pltpu.VMEM_SHARED((NPAD, D), jnp.float32)]
        + [pltpu.SemaphoreType.DMA] * NBUF
    )

    def body(vals_hbm, srcp_hbm, dstp_hbm, p_hbm, *rest):
        sidxa = rest[0]
        didxa = rest[1]
        rows = rest[2 : 2 + NBUF]
        zrow = rest[2 + NBUF]
        acc = rest[3 + NBUF]
        sems = rest[4 + NBUF : 4 + 2 * NBUF]

        cid = lax.axis_index("c")
        sid = lax.axis_index("s")
        cbase = (sid * NC + cid) * CPT

        def zb(i, carry):
            for j in range(D // 16):
                zrow[i, pl.ds(j * 16, 16)] = jnp.zeros((16,), jnp.float32)
            return carry

        with jax.named_scope("zinit"):
            lax.fori_loop(0, 16, zb, 0)

        def zc(k, carry):
            pltpu.sync_copy(zrow, acc.at[pl.ds(sid * ROWS_PER_TILE + k * 16, 16)])
            return carry

        with jax.named_scope("zcopy"):
            lax.fori_loop(0, ROWS_PER_TILE // 16, zc, 0)

        plsc.subcore_barrier()

        def issue(b, ci):
            pltpu.async_copy(vals_hbm.at[sidxa.at[ci]], rows[b], sems[b])

        def drain(b, ci):
            pltpu.make_async_copy(vals_hbm.at[sidxa.at[ci]], rows[b], sems[b]).wait()
            pltpu.sync_copy(rows[b], acc.at[didxa.at[ci]], add=True)

        with jax.named_scope("mainloop"):
            for h in range(CPT // HALF):
                hb = cbase + h * HALF
                pltpu.sync_copy(srcp_hbm.at[pl.ds(hb, HALF)], sidxa)
                pltpu.sync_copy(dstp_hbm.at[pl.ds(hb, HALF)], didxa)
                for b in range(NBUF):
                    issue(b, b)

                def step(g, carry):
                    for b in range(NBUF):
                        ci = g * NBUF + b
                        drain(b, ci)
                        issue(b, ci + NBUF)
                    return carry

                lax.fori_loop(0, HALF // NBUF - 1, step, 0)
                for b in range(NBUF):
                    drain(b, HALF - NBUF + b)

        with jax.named_scope("outbar"):
            plsc.subcore_barrier()

        with jax.named_scope("outcopy"):
            rbase = sid * ROWS_PER_TILE
            pltpu.sync_copy(
                acc.at[pl.ds(rbase, ROWS_PER_TILE)],
                p_hbm.at[pl.ds(cid * NPAD + rbase, ROWS_PER_TILE)],
            )

    return pl.kernel(
        body,
        out_type=(jax.ShapeDtypeStruct((NC * NPAD, D), jnp.float32),),
        mesh=mesh,
        scratch_types=scratch,
        compiler_params=pltpu.CompilerParams(needs_layout_passes=False),
    )


def _make_counts():
    mesh = plsc.VectorSubcoreMesh(core_axis_name="c", subcore_axis_name="s")

    scratch = [
        pltpu.VMEM((CPT, C), jnp.int32),
        pltpu.VMEM((NPAD,), jnp.float32),
    ]

    def body(dstp_hbm, cnt_hbm, didxa, cntv):
        cid = lax.axis_index("c")
        sid = lax.axis_index("s")
        wid = sid * NC + cid

        def zcnt(i, carry):
            cntv[pl.ds(i * 16, 16)] = jnp.zeros((16,), jnp.float32)
            return carry

        with jax.named_scope("zcnt"):
            lax.fori_loop(0, NPAD // 16, zcnt, 0)

        with jax.named_scope("cntidx"):
            pltpu.sync_copy(dstp_hbm.at[pl.ds(wid * CPT, CPT)], didxa)

        ones16 = jnp.ones((16,), jnp.float32)

        def cstep(ci, carry):
            for j in range(C // 16):
                dvec = didxa[ci, pl.ds(j * 16, 16)]
                plsc.addupdate_scatter(cntv, [dvec], ones16)
            return carry

        with jax.named_scope("cntloop"):
            lax.fori_loop(0, CPT, cstep, 0)

        with jax.named_scope("cntout"):
            pltpu.sync_copy(cntv, cnt_hbm.at[wid])

    return pl.kernel(
        body,
        out_type=(jax.ShapeDtypeStruct((NW, NPAD), jnp.float32),),
        mesh=mesh,
        scratch_types=scratch,
        compiler_params=pltpu.CompilerParams(needs_layout_passes=False),
    )


_SEG = _make_seg()
_COUNTS = _make_counts()

BLK = 1024


def _combine_body(p0_ref, p1_ref, cnt_ref, v_ref, wl_ref, wr_ref, b_ref, o_ref, *, act):
    cnt = jnp.sum(cnt_ref[...], axis=0)
    recip = 1.0 / jnp.maximum(cnt, 1.0)
    agg = (p0_ref[...] + p1_ref[...]) * recip[:, None]
    r = (
        jnp.dot(agg, wl_ref[...], preferred_element_type=jnp.float32)
        + jnp.dot(v_ref[...], wr_ref[...], preferred_element_type=jnp.float32)
        + b_ref[...]
    )
    o_ref[...] = act(r)


def _make_combine(act):
    return pl.pallas_call(
        functools.partial(_combine_body, act=act),
        grid=(NPAD // BLK,),
        in_specs=[
            pl.BlockSpec((BLK, D), lambda i: (i, 0)),
            pl.BlockSpec((BLK, D), lambda i: (i + NPAD // BLK, 0)),
            pl.BlockSpec((NW, BLK), lambda i: (0, i)),
            pl.BlockSpec((BLK, D), lambda i: (i, 0)),
            pl.BlockSpec((D, D), lambda i: (0, 0)),
            pl.BlockSpec((D, D), lambda i: (0, 0)),
            pl.BlockSpec((1, D), lambda i: (0, 0)),
        ],
        out_specs=pl.BlockSpec((BLK, D), lambda i: (i, 0)),
        out_shape=jax.ShapeDtypeStruct((NPAD, D), jnp.float32),
    )


_COMBINE_RELU = _make_combine(jax.nn.relu)
_COMBINE_SIGMOID = _make_combine(jax.nn.sigmoid)


def kernel(x, edge_index, W1_l, b1, W1_r, W2_l, b2, W2_r):
    src = edge_index[0].astype(jnp.int32)
    dst = edge_index[1].astype(jnp.int32)
    pad_e = EPAD - N_EDGES
    pad_iota = jnp.arange(pad_e, dtype=jnp.int32)
    srcp = jnp.concatenate([src, pad_iota % N_NODES]).reshape(NCHUNKS, C)
    dstp = jnp.concatenate(
        [dst, N_NODES + pad_iota % (NPAD - N_NODES)]
    ).reshape(NCHUNKS, C)
    xp = jnp.concatenate(
        [x.astype(jnp.float32), jnp.zeros((NPAD - N_NODES, D), jnp.float32)]
    )

    (p1,) = _SEG(xp, srcp, dstp)
    (cnt,) = _COUNTS(dstp)
    h = _COMBINE_RELU(p1, p1, cnt, xp, W1_l.T, W1_r.T, b1.reshape(1, D))
    (p2,) = _SEG(h, srcp, dstp)
    out = _COMBINE_SIGMOID(p2, p2, cnt, h, W2_l.T, W2_r.T, b2.reshape(1, D))
    return out[:N_NODES]
